# baseline (speedup 1.0000x reference)
import functools

import jax
import jax.numpy as jnp
from jax import lax
from jax.experimental import pallas as pl
from jax.experimental.pallas import tpu as pltpu
from jax.experimental.pallas import tpu_sc as plsc

_K = 8192
_D = 32
_N = 8192
_BT = 512
_NBLK = _N // _BT
_COMMIT = 0.25
_ENT_RATIO = 0.1
_TEMP = 0.01


def _vq_block(x_ref, cb_ref, a2_ref, b2_ref, idx_ref, loss_ref,
              acc_ref, sums_ref):
    i = pl.program_id(0)

    @pl.when(i == 0)
    def _init():
        acc_ref[...] = jnp.zeros_like(acc_ref)
        sums_ref[0] = 0.0
        sums_ref[1] = 0.0

    xb = x_ref[...]
    cb = cb_ref[...]
    a2 = a2_ref[...]
    b2 = b2_ref[...]
    ab2 = jax.lax.dot_general(xb * (-2.0), cb, (((1,), (1,)), ((), ())),
                              preferred_element_type=jnp.float32)
    d = (a2 + ab2) + b2

    dmin = jnp.min(d, axis=1, keepdims=True)
    idx = jnp.argmin(d, axis=1).astype(jnp.int32)
    idx_ref[0, 0, :] = idx

    z = ((dmin - d) * (1.0 / _TEMP)).astype(jnp.bfloat16)
    e = jnp.exp(z)
    ez = e * z
    ones_col = jnp.ones((_K, 1), jnp.bfloat16)
    s = jax.lax.dot_general(e, ones_col, (((1,), (0,)), ((), ())),
                            preferred_element_type=jnp.float32)
    rs = 1.0 / s
    rs_row = rs.reshape(1, _BT).astype(jnp.bfloat16)
    acc_ref[...] += jax.lax.dot_general(
        rs_row, e, (((1,), (0,)), ((), ())),
        preferred_element_type=jnp.float32)
    pz_vec = jax.lax.dot_general(
        rs_row, ez, (((1,), (0,)), ((), ())),
        preferred_element_type=jnp.float32)
    sums_ref[0] += jnp.sum(pz_vec) - jnp.sum(jnp.log(s))
    sums_ref[1] += jnp.sum(dmin)

    @pl.when(i == _NBLK - 1)
    def _fin():
        avg = acc_ref[...] / _N
        avg_ent = -jnp.sum(avg * jnp.log(avg + 1e-5))
        sample_ent = -(sums_ref[0] / _N)
        mse = sums_ref[1] / (_N * _D)
        loss_ref[0, 0] = (mse * _COMMIT + mse
                          + _ENT_RATIO * (sample_ent - avg_ent))


def _make_sc_gather():
    info = plsc.get_sparse_core_info()
    nw = info.num_cores * info.num_subcores
    b_per_w = _N // nw
    mesh = plsc.VectorSubcoreMesh(core_axis_name="c", subcore_axis_name="s")

    @functools.partial(
        pl.kernel, mesh=mesh,
        out_type=jax.ShapeDtypeStruct((_N, 128), jnp.float32),
        scratch_types=[
            pltpu.VMEM((b_per_w,), jnp.int32),
            pltpu.VMEM((b_per_w, 128), jnp.float32),
            pltpu.SemaphoreType.DMA,
        ],
    )
    def gather_k(table_hbm, idx_hbm, out_hbm, idx_v, rows_v, sem):
        wid = lax.axis_index("s") * info.num_cores + lax.axis_index("c")
        base = wid * b_per_w
        pltpu.sync_copy(idx_hbm.at[pl.ds(base, b_per_w)], idx_v)
        pltpu.async_copy(table_hbm.at[idx_v], rows_v, sem).wait()
        pltpu.sync_copy(rows_v, out_hbm.at[pl.ds(base, b_per_w)])

    return gather_k


_sc_gather = _make_sc_gather()


def kernel(x, codebook):
    codebook = jnp.asarray(codebook, dtype=jnp.float32)
    xf = jnp.reshape(x, (-1, _D))
    a2 = jnp.sum(xf ** 2, axis=1, keepdims=True)
    b2 = jnp.sum(codebook.T ** 2, axis=0, keepdims=True)
    idx, loss = pl.pallas_call(
        _vq_block,
        grid=(_NBLK,),
        in_specs=[
            pl.BlockSpec((_BT, _D), lambda i: (i, 0)),
            pl.BlockSpec((_K, _D), lambda i: (0, 0)),
            pl.BlockSpec((_BT, 1), lambda i: (i, 0)),
            pl.BlockSpec((1, _K), lambda i: (0, 0)),
        ],
        out_specs=[
            pl.BlockSpec((1, 1, _BT), lambda i: (i, 0, 0)),
            pl.BlockSpec(memory_space=pltpu.SMEM, block_shape=(1, 1),
                         index_map=lambda i: (0, 0)),
        ],
        out_shape=[
            jax.ShapeDtypeStruct((_NBLK, 1, _BT), jnp.int32),
            jax.ShapeDtypeStruct((1, 1), jnp.float32),
        ],
        scratch_shapes=[
            pltpu.VMEM((1, _K), jnp.float32),
            pltpu.SMEM((2,), jnp.float32),
        ],
    )(xf, codebook, a2, b2)
    idx_flat = jnp.reshape(idx, (_N,))
    cb_pad = jnp.pad(codebook, ((0, 0), (0, 128 - _D)))
    quant = _sc_gather(cb_pad, idx_flat)[:, :_D]
    quantized = jnp.reshape(quant, x.shape)
    encoding_indices = jnp.reshape(idx, x.shape[:-1])
    return quantized, loss[0, 0], encoding_indices


# speedup vs baseline: 1.1185x; 1.1185x over previous
import functools

import jax
import jax.numpy as jnp
from jax import lax
from jax.experimental import pallas as pl
from jax.experimental.pallas import tpu as pltpu
from jax.experimental.pallas import tpu_sc as plsc

_K = 8192
_D = 32
_N = 8192
_BT = 128
_NBLK = _N // _BT
_COMMIT = 0.25
_ENT_RATIO = 0.1
_TEMP = 0.01


def _vq_block(x_ref, cb_ref, a2_ref, b2_ref, idx_ref, loss_ref,
              acc_ref, sums_ref):
    i = pl.program_id(0)

    @pl.when(i == 0)
    def _init():
        acc_ref[...] = jnp.zeros_like(acc_ref)
        sums_ref[0] = 0.0
        sums_ref[1] = 0.0

    xb = x_ref[...]
    cb = cb_ref[...]
    a2 = a2_ref[...]
    b2 = b2_ref[...]
    ab2 = jax.lax.dot_general(xb * (-2.0), cb, (((1,), (1,)), ((), ())),
                              preferred_element_type=jnp.float32)
    d = (a2 + ab2) + b2

    dmin = jnp.min(d, axis=1, keepdims=True)
    idx = jnp.argmin(d, axis=1).astype(jnp.int32)
    idx_ref[0, 0, :] = idx

    z = ((dmin - d) * (1.0 / _TEMP)).astype(jnp.bfloat16)
    e = jnp.exp(z)
    ez = e * z
    ones_col = jnp.ones((_K, 1), jnp.bfloat16)
    s = jax.lax.dot_general(e, ones_col, (((1,), (0,)), ((), ())),
                            preferred_element_type=jnp.float32)
    rs = 1.0 / s
    rs_row = rs.reshape(1, _BT).astype(jnp.bfloat16)
    acc_ref[...] += jax.lax.dot_general(
        rs_row, e, (((1,), (0,)), ((), ())),
        preferred_element_type=jnp.float32)
    pz_vec = jax.lax.dot_general(
        rs_row, ez, (((1,), (0,)), ((), ())),
        preferred_element_type=jnp.float32)
    sums_ref[0] += jnp.sum(pz_vec) - jnp.sum(jnp.log(s))
    sums_ref[1] += jnp.sum(dmin)

    @pl.when(i == _NBLK - 1)
    def _fin():
        avg = acc_ref[...] / _N
        avg_ent = -jnp.sum(avg * jnp.log(avg + 1e-5))
        sample_ent = -(sums_ref[0] / _N)
        mse = sums_ref[1] / (_N * _D)
        loss_ref[0, 0] = (mse * _COMMIT + mse
                          + _ENT_RATIO * (sample_ent - avg_ent))


def _make_sc_gather():
    info = plsc.get_sparse_core_info()
    nw = info.num_cores * info.num_subcores
    b_per_w = _N // nw
    mesh = plsc.VectorSubcoreMesh(core_axis_name="c", subcore_axis_name="s")

    @functools.partial(
        pl.kernel, mesh=mesh,
        out_type=jax.ShapeDtypeStruct((_N, 128), jnp.float32),
        scratch_types=[
            pltpu.VMEM((b_per_w,), jnp.int32),
            pltpu.VMEM((b_per_w, 128), jnp.float32),
            pltpu.SemaphoreType.DMA,
        ],
    )
    def gather_k(table_hbm, idx_hbm, out_hbm, idx_v, rows_v, sem):
        wid = lax.axis_index("s") * info.num_cores + lax.axis_index("c")
        base = wid * b_per_w
        pltpu.sync_copy(idx_hbm.at[pl.ds(base, b_per_w)], idx_v)
        pltpu.async_copy(table_hbm.at[idx_v], rows_v, sem).wait()
        pltpu.sync_copy(rows_v, out_hbm.at[pl.ds(base, b_per_w)])

    return gather_k


_sc_gather = _make_sc_gather()


def kernel(x, codebook):
    codebook = jnp.asarray(codebook, dtype=jnp.float32)
    xf = jnp.reshape(x, (-1, _D))
    a2 = jnp.sum(xf ** 2, axis=1, keepdims=True)
    b2 = jnp.sum(codebook.T ** 2, axis=0, keepdims=True)
    idx, loss = pl.pallas_call(
        _vq_block,
        grid=(_NBLK,),
        in_specs=[
            pl.BlockSpec((_BT, _D), lambda i: (i, 0)),
            pl.BlockSpec((_K, _D), lambda i: (0, 0)),
            pl.BlockSpec((_BT, 1), lambda i: (i, 0)),
            pl.BlockSpec((1, _K), lambda i: (0, 0)),
        ],
        out_specs=[
            pl.BlockSpec((1, 1, _BT), lambda i: (i, 0, 0)),
            pl.BlockSpec(memory_space=pltpu.SMEM, block_shape=(1, 1),
                         index_map=lambda i: (0, 0)),
        ],
        out_shape=[
            jax.ShapeDtypeStruct((_NBLK, 1, _BT), jnp.int32),
            jax.ShapeDtypeStruct((1, 1), jnp.float32),
        ],
        scratch_shapes=[
            pltpu.VMEM((1, _K), jnp.float32),
            pltpu.SMEM((2,), jnp.float32),
        ],
    )(xf, codebook, a2, b2)
    idx_flat = jnp.reshape(idx, (_N,))
    cb_pad = jnp.pad(codebook, ((0, 0), (0, 128 - _D)))
    quant = _sc_gather(cb_pad, idx_flat)[:, :_D]
    quantized = jnp.reshape(quant, x.shape)
    encoding_indices = jnp.reshape(idx, x.shape[:-1])
    return quantized, loss[0, 0], encoding_indices


# speedup vs baseline: 1.2389x; 1.1076x over previous
import functools

import jax
import jax.numpy as jnp
from jax import lax
from jax.experimental import pallas as pl
from jax.experimental.pallas import tpu as pltpu
from jax.experimental.pallas import tpu_sc as plsc

_K = 8192
_D = 32
_N = 8192
_BT = 256
_NBLK = _N // _BT
_COMMIT = 0.25
_ENT_RATIO = 0.1
_TEMP = 0.01


def _vq_block(x_ref, cb_ref, a2_ref, b2_ref, idx_ref, loss_ref,
              acc_ref, sums_ref):
    i = pl.program_id(0)

    @pl.when(i == 0)
    def _init():
        acc_ref[...] = jnp.zeros_like(acc_ref)
        sums_ref[0] = 0.0
        sums_ref[1] = 0.0

    xb = x_ref[...]
    cb = cb_ref[...]
    a2 = a2_ref[...]
    b2 = b2_ref[...]
    ab2 = jax.lax.dot_general(xb * (-2.0), cb, (((1,), (1,)), ((), ())),
                              preferred_element_type=jnp.float32)
    d = (a2 + ab2) + b2

    dmin = jnp.min(d, axis=1, keepdims=True)
    idx = jnp.argmin(d, axis=1).astype(jnp.int32)
    idx_ref[0, 0, :] = idx

    z = (dmin - d).astype(jnp.bfloat16) * jnp.bfloat16(1.0 / _TEMP)
    e = jnp.exp(z)
    ez = e * z
    ones_col = jnp.ones((_K, 1), jnp.bfloat16)
    s = jax.lax.dot_general(e, ones_col, (((1,), (0,)), ((), ())),
                            preferred_element_type=jnp.float32)
    rs = 1.0 / s
    rs_row = rs.reshape(1, _BT).astype(jnp.bfloat16)
    acc_ref[...] += jax.lax.dot_general(
        rs_row, e, (((1,), (0,)), ((), ())),
        preferred_element_type=jnp.float32)
    pz_vec = jax.lax.dot_general(
        rs_row, ez, (((1,), (0,)), ((), ())),
        preferred_element_type=jnp.float32)
    sums_ref[0] += jnp.sum(pz_vec) - jnp.sum(jnp.log(s))
    sums_ref[1] += jnp.sum(dmin)

    @pl.when(i == _NBLK - 1)
    def _fin():
        avg = acc_ref[...] / _N
        avg_ent = -jnp.sum(avg * jnp.log(avg + 1e-5))
        sample_ent = -(sums_ref[0] / _N)
        mse = sums_ref[1] / (_N * _D)
        loss_ref[0, 0] = (mse * _COMMIT + mse
                          + _ENT_RATIO * (sample_ent - avg_ent))


def _make_sc_gather():
    info = plsc.get_sparse_core_info()
    nw = info.num_cores * info.num_subcores
    b_per_w = _N // nw
    mesh = plsc.VectorSubcoreMesh(core_axis_name="c", subcore_axis_name="s")

    @functools.partial(
        pl.kernel, mesh=mesh,
        out_type=jax.ShapeDtypeStruct((_N, 128), jnp.float32),
        scratch_types=[
            pltpu.VMEM((b_per_w,), jnp.int32),
            pltpu.VMEM((b_per_w, 128), jnp.float32),
            pltpu.SemaphoreType.DMA,
        ],
    )
    def gather_k(table_hbm, idx_hbm, out_hbm, idx_v, rows_v, sem):
        wid = lax.axis_index("s") * info.num_cores + lax.axis_index("c")
        base = wid * b_per_w
        pltpu.sync_copy(idx_hbm.at[pl.ds(base, b_per_w)], idx_v)
        pltpu.async_copy(table_hbm.at[idx_v], rows_v, sem).wait()
        pltpu.sync_copy(rows_v, out_hbm.at[pl.ds(base, b_per_w)])

    return gather_k


_sc_gather = _make_sc_gather()


def kernel(x, codebook):
    codebook = jnp.asarray(codebook, dtype=jnp.float32)
    xf = jnp.reshape(x, (-1, _D))
    a2 = jnp.sum(xf ** 2, axis=1, keepdims=True)
    b2 = jnp.sum(codebook.T ** 2, axis=0, keepdims=True)
    idx, loss = pl.pallas_call(
        _vq_block,
        grid=(_NBLK,),
        in_specs=[
            pl.BlockSpec((_BT, _D), lambda i: (i, 0)),
            pl.BlockSpec((_K, _D), lambda i: (0, 0)),
            pl.BlockSpec((_BT, 1), lambda i: (i, 0)),
            pl.BlockSpec((1, _K), lambda i: (0, 0)),
        ],
        out_specs=[
            pl.BlockSpec((1, 1, _BT), lambda i: (i, 0, 0)),
            pl.BlockSpec(memory_space=pltpu.SMEM, block_shape=(1, 1),
                         index_map=lambda i: (0, 0)),
        ],
        out_shape=[
            jax.ShapeDtypeStruct((_NBLK, 1, _BT), jnp.int32),
            jax.ShapeDtypeStruct((1, 1), jnp.float32),
        ],
        scratch_shapes=[
            pltpu.VMEM((1, _K), jnp.float32),
            pltpu.SMEM((2,), jnp.float32),
        ],
    )(xf, codebook, a2, b2)
    idx_flat = jnp.reshape(idx, (_N,))
    cb_pad = jnp.pad(codebook, ((0, 0), (0, 128 - _D)))
    quant = _sc_gather(cb_pad, idx_flat)[:, :_D]
    quantized = jnp.reshape(quant, x.shape)
    encoding_indices = jnp.reshape(idx, x.shape[:-1])
    return quantized, loss[0, 0], encoding_indices


# speedup vs baseline: 1.3074x; 1.0553x over previous
import functools

import jax
import jax.numpy as jnp
from jax import lax
from jax.experimental import pallas as pl
from jax.experimental.pallas import tpu as pltpu
from jax.experimental.pallas import tpu_sc as plsc

_K = 8192
_D = 32
_N = 8192
_BT = 256
_NBLK = _N // _BT
_COMMIT = 0.25
_ENT_RATIO = 0.1
_TEMP = 0.01


def _vq_block(x_ref, cb_ref, a2_ref, b2_ref, idx_ref, loss_ref,
              acc_ref, sums_ref):
    i = pl.program_id(0)

    @pl.when(i == 0)
    def _init():
        acc_ref[...] = jnp.zeros_like(acc_ref)
        sums_ref[0] = 0.0
        sums_ref[1] = 0.0

    xb = x_ref[...]
    cb = cb_ref[...]
    a2 = a2_ref[...]
    b2 = b2_ref[...]
    ab2 = jax.lax.dot_general(xb * (-2.0), cb, (((1,), (1,)), ((), ())),
                              preferred_element_type=jnp.float32)
    d = (a2 + ab2) + b2

    dmin = jnp.min(d, axis=1, keepdims=True)
    iota = jax.lax.broadcasted_iota(jnp.int32, d.shape, 1)
    idx = jnp.min(jnp.where(d == dmin, iota, _K), axis=1).astype(jnp.int32)
    idx_ref[0, 0, :] = idx

    z = (dmin - d).astype(jnp.bfloat16) * jnp.bfloat16(1.0 / _TEMP)
    e = jnp.exp(z)
    ez = e * z
    ones_col = jnp.ones((_K, 1), jnp.bfloat16)
    s = jax.lax.dot_general(e, ones_col, (((1,), (0,)), ((), ())),
                            preferred_element_type=jnp.float32)
    rs = 1.0 / s
    rs_row = rs.reshape(1, _BT).astype(jnp.bfloat16)
    acc_ref[...] += jax.lax.dot_general(
        rs_row, e, (((1,), (0,)), ((), ())),
        preferred_element_type=jnp.float32)
    pz_vec = jax.lax.dot_general(
        rs_row, ez, (((1,), (0,)), ((), ())),
        preferred_element_type=jnp.float32)
    sums_ref[0] += jnp.sum(pz_vec) - jnp.sum(jnp.log(s))
    sums_ref[1] += jnp.sum(dmin)

    @pl.when(i == _NBLK - 1)
    def _fin():
        avg = acc_ref[...] / _N
        avg_ent = -jnp.sum(avg * jnp.log(avg + 1e-5))
        sample_ent = -(sums_ref[0] / _N)
        mse = sums_ref[1] / (_N * _D)
        loss_ref[0, 0] = (mse * _COMMIT + mse
                          + _ENT_RATIO * (sample_ent - avg_ent))


def _make_sc_gather():
    info = plsc.get_sparse_core_info()
    nw = info.num_cores * info.num_subcores
    b_per_w = _N // nw
    mesh = plsc.VectorSubcoreMesh(core_axis_name="c", subcore_axis_name="s")

    @functools.partial(
        pl.kernel, mesh=mesh,
        out_type=jax.ShapeDtypeStruct((_N, 128), jnp.float32),
        scratch_types=[
            pltpu.VMEM((b_per_w,), jnp.int32),
            pltpu.VMEM((b_per_w, 128), jnp.float32),
            pltpu.SemaphoreType.DMA,
        ],
    )
    def gather_k(table_hbm, idx_hbm, out_hbm, idx_v, rows_v, sem):
        wid = lax.axis_index("s") * info.num_cores + lax.axis_index("c")
        base = wid * b_per_w
        pltpu.sync_copy(idx_hbm.at[pl.ds(base, b_per_w)], idx_v)
        pltpu.async_copy(table_hbm.at[idx_v], rows_v, sem).wait()
        pltpu.sync_copy(rows_v, out_hbm.at[pl.ds(base, b_per_w)])

    return gather_k


_sc_gather = _make_sc_gather()


def kernel(x, codebook):
    codebook = jnp.asarray(codebook, dtype=jnp.float32)
    xf = jnp.reshape(x, (-1, _D))
    a2 = jnp.sum(xf ** 2, axis=1, keepdims=True)
    b2 = jnp.sum(codebook.T ** 2, axis=0, keepdims=True)
    idx, loss = pl.pallas_call(
        _vq_block,
        grid=(_NBLK,),
        in_specs=[
            pl.BlockSpec((_BT, _D), lambda i: (i, 0)),
            pl.BlockSpec((_K, _D), lambda i: (0, 0)),
            pl.BlockSpec((_BT, 1), lambda i: (i, 0)),
            pl.BlockSpec((1, _K), lambda i: (0, 0)),
        ],
        out_specs=[
            pl.BlockSpec((1, 1, _BT), lambda i: (i, 0, 0)),
            pl.BlockSpec(memory_space=pltpu.SMEM, block_shape=(1, 1),
                         index_map=lambda i: (0, 0)),
        ],
        out_shape=[
            jax.ShapeDtypeStruct((_NBLK, 1, _BT), jnp.int32),
            jax.ShapeDtypeStruct((1, 1), jnp.float32),
        ],
        scratch_shapes=[
            pltpu.VMEM((1, _K), jnp.float32),
            pltpu.SMEM((2,), jnp.float32),
        ],
    )(xf, codebook, a2, b2)
    idx_flat = jnp.reshape(idx, (_N,))
    cb_pad = jnp.pad(codebook, ((0, 0), (0, 128 - _D)))
    quant = _sc_gather(cb_pad, idx_flat)[:, :_D]
    quantized = jnp.reshape(quant, x.shape)
    encoding_indices = jnp.reshape(idx, x.shape[:-1])
    return quantized, loss[0, 0], encoding_indices
